# baseline (speedup 1.0000x reference)
import functools

import jax
import jax.numpy as jnp
from jax import lax
from jax.experimental import pallas as pl
from jax.experimental.pallas import tpu as pltpu
from jax.experimental.pallas import tpu_sc as plsc

N = 64
D = 128
L = 16
NCHUNK = N // L
NC = 2
NS = 16
NW = NC * NS
APW = N // NW
NEG = -1e30

_mesh = plsc.VectorSubcoreMesh(
    core_axis_name="c", subcore_axis_name="s", num_cores=NC, num_subcores=NS
)


@functools.partial(
    pl.kernel,
    out_type=jax.ShapeDtypeStruct((NC, L), jnp.float32),
    mesh=_mesh,
    compiler_params=pltpu.CompilerParams(needs_layout_passes=False),
    scratch_types=[
        pltpu.VMEM((D * N,), jnp.float32),
        pltpu.VMEM((N,), jnp.int32),
        pltpu.VMEM((L,), jnp.float32),
        pltpu.VMEM((APW * N,), jnp.float32),
        pltpu.VMEM((L,), jnp.float32),
        pltpu.VMEM((1, 2 * L), jnp.float32),
        pltpu.VMEM((1, 2 * L), jnp.float32),
        pltpu.VMEM((1, L), jnp.float32),
        pltpu.VMEM((1,), jnp.int32),
        pltpu.VMEM_SHARED((1, 2 * L), jnp.float32),
        pltpu.SemaphoreType.DMA,
        pltpu.SemaphoreType.DMA,
    ],
)
def _triplet_sc(ef_hbm, lab_hbm, max_hbm, zidx_hbm, out_hbm,
                ef_v, lab_v, histf_v, r_v, max_v,
                part_v, red_v, wr_v, zidx_v, shared, sem_s, sem_b):
    c = lax.axis_index("c")
    s = lax.axis_index("s")
    wid = s * NC + c
    a0 = wid * APW
    a1 = a0 + 1

    h_ef = pltpu.async_copy(ef_hbm, ef_v, sem_b)
    h_lab = pltpu.async_copy(lab_hbm, lab_v, sem_s)
    h_max = pltpu.async_copy(max_hbm, max_v, sem_s)
    h_zi = pltpu.async_copy(zidx_hbm, zidx_v, sem_s)
    h_lab.wait()
    h_max.wait()
    h_zi.wait()

    iota = lax.iota(jnp.int32, L)
    zf = jnp.zeros((L,), jnp.float32)
    inv = 1.0 / max_v[...]

    @pl.when(s == 0)
    def _():
        red_v[0, pl.ds(0, L)] = zf
        red_v[0, pl.ds(L, L)] = zf
        pltpu.sync_copy(red_v, shared)

    hs = [jnp.zeros((L,), jnp.int32) for _ in range(4)]
    for j in range(N):
        lj = plsc.load_gather(lab_v, [jnp.full((L,), j, jnp.int32)])
        hs[j % 4] = hs[j % 4] + jnp.where(iota == lj, 1, 0).astype(jnp.int32)
    histf_v[...] = (hs[0] + hs[1] + hs[2] + hs[3]).astype(jnp.float32)

    h_ef.wait()

    @plsc.parallel_loop(0, D, unroll=2, carry=(zf,) * (2 * NCHUNK))
    def p1_accs(d, accs):
        base = d * N
        ea0 = plsc.load_gather(ef_v, [jnp.full((L,), base + a0, jnp.int32)])
        ea1 = plsc.load_gather(ef_v, [jnp.full((L,), base + a1, jnp.int32)])
        out = []
        for q in range(NCHUNK):
            col = ef_v[pl.ds(base + q * L, L)]
            d0 = col - ea0
            d1 = col - ea1
            out.append(accs[q] + d0 * d0)
            out.append(accs[NCHUNK + q] + d1 * d1)
        return tuple(out[0::2]) + tuple(out[1::2])

    for q in range(NCHUNK):
        r_v[pl.ds(q * L, L)] = p1_accs[q]
        r_v[pl.ds(N + q * L, L)] = p1_accs[NCHUNK + q]

    lab_c = [lab_v[pl.ds(q * L, L)] for q in range(NCHUNK)]
    lpf_c = [lc.astype(jnp.float32) for lc in lab_c]
    cdf_c = [N - plsc.load_gather(histf_v, [lab_c[q]]) for q in range(NCHUNK)]
    laf0 = plsc.load_gather(lab_v, [jnp.full((L,), a0, jnp.int32)]
                            ).astype(jnp.float32)
    laf1 = plsc.load_gather(lab_v, [jnp.full((L,), a1, jnp.int32)]
                            ).astype(jnp.float32)

    rp_eff0 = []
    rp_eff1 = []
    bfs_c = []
    cnt_corr = zf
    for q in range(NCHUNK):
        pc = iota + q * L
        valid = cdf_c[q] >= 2.0
        cond0 = valid & (pc != a0)
        cond1 = valid & (pc != a1)
        bf0 = jnp.where(cond0, 1.0, 0.0)
        bf1 = jnp.where(cond1, 1.0, 0.0)
        bfs_c.append(bf0 + bf1)
        rp0 = r_v[pl.ds(q * L, L)]
        rp1 = r_v[pl.ds(N + q * L, L)]
        rp_eff0.append(jnp.where(cond0, rp0, NEG))
        rp_eff1.append(jnp.where(cond1, rp1, NEG))
        t0 = lpf_c[q] - laf0
        t1 = lpf_c[q] - laf1
        cnt_corr = (cnt_corr - jnp.where(t0 == 0.0, 0.0, bf0)
                    - jnp.where(t1 == 0.0, 0.0, bf1))

    def chunk_terms(q, rn0, rn1, lnf):
        t = lpf_c[q] - lnf
        pen = jnp.abs(t) * inv
        pen_eff = jnp.where(t == 0.0, NEG, pen)
        v0 = jnp.maximum(rp_eff0[q] - rn0 + pen_eff, 0.0)
        v1 = jnp.maximum(rp_eff1[q] - rn1 + pen_eff, 0.0)
        cm = jnp.where(t == 0.0, 0.0, bfs_c[q])
        return v0, v1, cm

    @plsc.parallel_loop(0, N, unroll=4, carry=(zf,) * (3 * NCHUNK))
    def sums(n, carry):
        idx = jnp.full((L,), n, jnp.int32)
        rn0 = plsc.load_gather(r_v, [idx])
        rn1 = plsc.load_gather(r_v, [idx + N])
        lnf = plsc.load_gather(lab_v, [idx]).astype(jnp.float32)
        out = list(carry)
        for q in range(NCHUNK):
            v0, v1, cm = chunk_terms(q, rn0, rn1, lnf)
            out[q] = carry[q] + v0
            out[NCHUNK + q] = carry[NCHUNK + q] + v1
            out[2 * NCHUNK + q] = carry[2 * NCHUNK + q] + cm
        return tuple(out)

    ia0 = jnp.full((L,), a0, jnp.int32)
    ia1 = jnp.full((L,), a1, jnp.int32)
    ra00 = plsc.load_gather(r_v, [ia0])
    ra01 = plsc.load_gather(r_v, [ia0 + N])
    ra10 = plsc.load_gather(r_v, [ia1])
    ra11 = plsc.load_gather(r_v, [ia1 + N])
    acc_tot = zf
    cnt_tot = cnt_corr
    for q in range(NCHUNK):
        v0a, _, _ = chunk_terms(q, ra00, ra01, laf0)
        _, v1a, _ = chunk_terms(q, ra10, ra11, laf1)
        acc_tot = acc_tot + (sums[q] - v0a) + (sums[NCHUNK + q] - v1a)
        cnt_tot = cnt_tot + sums[2 * NCHUNK + q]

    part_v[0, pl.ds(0, L)] = acc_tot
    part_v[0, pl.ds(L, L)] = cnt_tot
    plsc.subcore_barrier()
    pltpu.sync_copy(part_v, shared.at[zidx_v], add=True)
    plsc.subcore_barrier()

    @pl.when(s == 0)
    def _():
        pltpu.sync_copy(shared, red_v)
        tsum = jnp.sum(red_v[0, pl.ds(0, L)])
        csum = jnp.sum(red_v[0, pl.ds(L, L)])
        wr_v[0, pl.ds(0, L)] = jnp.where(
            iota == 0, tsum, jnp.where(iota == 1, csum, 0.0))
        pltpu.sync_copy(wr_v, out_hbm.at[pl.ds(c, 1)])


def kernel(embeddings, target, max_score):
    ef = embeddings.T.reshape(-1)
    maxf = jnp.broadcast_to(
        jnp.asarray(max_score).astype(jnp.float32), (L,))
    zidx = jnp.zeros((1,), jnp.int32)
    out = _triplet_sc(ef, target, maxf, zidx)
    total = out[0, 0] + out[1, 0]
    cf = out[0, 1] + out[1, 1]
    mean = total / cf
    count = cf.astype(jnp.int32)
    return (mean, count)


# speedup vs baseline: 1.0118x; 1.0118x over previous
import functools

import jax
import jax.numpy as jnp
from jax import lax
from jax.experimental import pallas as pl
from jax.experimental.pallas import tpu as pltpu
from jax.experimental.pallas import tpu_sc as plsc

N = 64
D = 128
L = 16
NCHUNK = N // L
NC = 2
NS = 16
NW = NC * NS
APW = N // NW
NEG = -1e30

_mesh = plsc.VectorSubcoreMesh(
    core_axis_name="c", subcore_axis_name="s", num_cores=NC, num_subcores=NS
)


@functools.partial(
    pl.kernel,
    out_type=jax.ShapeDtypeStruct((NC, L), jnp.float32),
    mesh=_mesh,
    compiler_params=pltpu.CompilerParams(needs_layout_passes=False),
    scratch_types=[
        pltpu.VMEM((D * N,), jnp.float32),
        pltpu.VMEM((N,), jnp.int32),
        pltpu.VMEM((L,), jnp.float32),
        pltpu.VMEM((APW * N,), jnp.float32),
        pltpu.VMEM((L,), jnp.float32),
        pltpu.VMEM((1, 2 * L), jnp.float32),
        pltpu.VMEM((1, 2 * L), jnp.float32),
        pltpu.VMEM((1, L), jnp.float32),
        pltpu.VMEM((1,), jnp.int32),
        pltpu.VMEM_SHARED((1, 2 * L), jnp.float32),
        pltpu.SemaphoreType.DMA,
        pltpu.SemaphoreType.DMA,
    ],
)
def _triplet_sc(ef_hbm, lab_hbm, max_hbm, zidx_hbm, out_hbm,
                ef_v, lab_v, histf_v, r_v, max_v,
                part_v, red_v, wr_v, zidx_v, shared, sem_s, sem_b):
    c = lax.axis_index("c")
    s = lax.axis_index("s")
    wid = s * NC + c
    a0 = wid * APW
    a1 = a0 + 1

    h_ef = pltpu.async_copy(ef_hbm, ef_v, sem_b)
    h_lab = pltpu.async_copy(lab_hbm, lab_v, sem_s)
    h_max = pltpu.async_copy(max_hbm, max_v, sem_s)
    h_zi = pltpu.async_copy(zidx_hbm, zidx_v, sem_s)
    h_lab.wait()
    h_max.wait()
    h_zi.wait()

    iota = lax.iota(jnp.int32, L)
    zf = jnp.zeros((L,), jnp.float32)
    inv = 1.0 / max_v[...]

    @pl.when(s == 0)
    def _():
        red_v[0, pl.ds(0, L)] = zf
        red_v[0, pl.ds(L, L)] = zf
        pltpu.sync_copy(red_v, shared)

    hs = [jnp.zeros((L,), jnp.int32) for _ in range(4)]
    for j in range(N):
        lj = plsc.load_gather(lab_v, [jnp.full((L,), j, jnp.int32)])
        hs[j % 4] = hs[j % 4] + jnp.where(iota == lj, 1, 0).astype(jnp.int32)
    histf_v[...] = (hs[0] + hs[1] + hs[2] + hs[3]).astype(jnp.float32)

    h_ef.wait()

    @plsc.parallel_loop(0, D, unroll=2, carry=(zf,) * (2 * NCHUNK))
    def p1_accs(d, accs):
        base = d * N
        ea0 = plsc.load_gather(ef_v, [jnp.full((L,), base + a0, jnp.int32)])
        ea1 = plsc.load_gather(ef_v, [jnp.full((L,), base + a1, jnp.int32)])
        out = []
        for q in range(NCHUNK):
            col = ef_v[pl.ds(base + q * L, L)]
            d0 = col - ea0
            d1 = col - ea1
            out.append(accs[q] + d0 * d0)
            out.append(accs[NCHUNK + q] + d1 * d1)
        return tuple(out[0::2]) + tuple(out[1::2])

    for q in range(NCHUNK):
        r_v[pl.ds(q * L, L)] = p1_accs[q]
        r_v[pl.ds(N + q * L, L)] = p1_accs[NCHUNK + q]

    lab_c = [lab_v[pl.ds(q * L, L)] for q in range(NCHUNK)]
    lpf_c = [lc.astype(jnp.float32) for lc in lab_c]
    cdf_c = [N - plsc.load_gather(histf_v, [lab_c[q]]) for q in range(NCHUNK)]
    laf0 = plsc.load_gather(lab_v, [jnp.full((L,), a0, jnp.int32)]
                            ).astype(jnp.float32)
    laf1 = plsc.load_gather(lab_v, [jnp.full((L,), a1, jnp.int32)]
                            ).astype(jnp.float32)

    rp_eff0 = []
    rp_eff1 = []
    bfs_c = []
    cnt_corr = zf
    for q in range(NCHUNK):
        pc = iota + q * L
        valid = cdf_c[q] >= 2.0
        cond0 = valid & (pc != a0)
        cond1 = valid & (pc != a1)
        bf0 = jnp.where(cond0, 1.0, 0.0)
        bf1 = jnp.where(cond1, 1.0, 0.0)
        bfs_c.append(bf0 + bf1)
        rp0 = r_v[pl.ds(q * L, L)]
        rp1 = r_v[pl.ds(N + q * L, L)]
        rp_eff0.append(jnp.where(cond0, rp0, NEG))
        rp_eff1.append(jnp.where(cond1, rp1, NEG))
        t0 = lpf_c[q] - laf0
        t1 = lpf_c[q] - laf1
        cnt_corr = (cnt_corr - jnp.where(t0 == 0.0, 0.0, bf0)
                    - jnp.where(t1 == 0.0, 0.0, bf1))

    def chunk_terms(q, rn0, rn1, lnf):
        t = lpf_c[q] - lnf
        pen = jnp.abs(t) * inv
        pen_eff = jnp.where(t == 0.0, NEG, pen)
        v0 = jnp.maximum(rp_eff0[q] - rn0 + pen_eff, 0.0)
        v1 = jnp.maximum(rp_eff1[q] - rn1 + pen_eff, 0.0)
        cm = jnp.where(t == 0.0, 0.0, bfs_c[q])
        return v0, v1, cm

    @plsc.parallel_loop(0, N, unroll=2, carry=(zf,) * (3 * NCHUNK))
    def sums(n, carry):
        idx = jnp.full((L,), n, jnp.int32)
        rn0 = plsc.load_gather(r_v, [idx])
        rn1 = plsc.load_gather(r_v, [idx + N])
        lnf = plsc.load_gather(lab_v, [idx]).astype(jnp.float32)
        out = list(carry)
        for q in range(NCHUNK):
            v0, v1, cm = chunk_terms(q, rn0, rn1, lnf)
            out[q] = carry[q] + v0
            out[NCHUNK + q] = carry[NCHUNK + q] + v1
            out[2 * NCHUNK + q] = carry[2 * NCHUNK + q] + cm
        return tuple(out)

    ia0 = jnp.full((L,), a0, jnp.int32)
    ia1 = jnp.full((L,), a1, jnp.int32)
    ra00 = plsc.load_gather(r_v, [ia0])
    ra01 = plsc.load_gather(r_v, [ia0 + N])
    ra10 = plsc.load_gather(r_v, [ia1])
    ra11 = plsc.load_gather(r_v, [ia1 + N])
    acc_tot = zf
    cnt_tot = cnt_corr
    for q in range(NCHUNK):
        v0a, _, _ = chunk_terms(q, ra00, ra01, laf0)
        _, v1a, _ = chunk_terms(q, ra10, ra11, laf1)
        acc_tot = acc_tot + (sums[q] - v0a) + (sums[NCHUNK + q] - v1a)
        cnt_tot = cnt_tot + sums[2 * NCHUNK + q]

    part_v[0, pl.ds(0, L)] = acc_tot
    part_v[0, pl.ds(L, L)] = cnt_tot
    plsc.subcore_barrier()
    pltpu.sync_copy(part_v, shared.at[zidx_v], add=True)
    plsc.subcore_barrier()

    @pl.when(s == 0)
    def _():
        pltpu.sync_copy(shared, red_v)
        tsum = jnp.sum(red_v[0, pl.ds(0, L)])
        csum = jnp.sum(red_v[0, pl.ds(L, L)])
        wr_v[0, pl.ds(0, L)] = jnp.where(
            iota == 0, tsum, jnp.where(iota == 1, csum, 0.0))
        pltpu.sync_copy(wr_v, out_hbm.at[pl.ds(c, 1)])


def kernel(embeddings, target, max_score):
    ef = embeddings.T.reshape(-1)
    maxf = jnp.broadcast_to(
        jnp.asarray(max_score).astype(jnp.float32), (L,))
    zidx = jnp.zeros((1,), jnp.int32)
    out = _triplet_sc(ef, target, maxf, zidx)
    total = out[0, 0] + out[1, 0]
    cf = out[0, 1] + out[1, 1]
    mean = total / cf
    count = cf.astype(jnp.int32)
    return (mean, count)
